# baseline (speedup 1.0000x reference)
import functools

import jax
import jax.numpy as jnp
from jax import lax
from jax.experimental import pallas as pl
from jax.experimental.pallas import tpu as pltpu
from jax.experimental.pallas import tpu_sc as plsc

VOCAB = 1000000
EMBED_DIM = 32
BATCH = 4096
HIST = 200

_INFO = plsc.get_sparse_core_info()
NC = _INFO.num_cores
NS = _INFO.num_subcores
NW = NC * NS

B_TOTAL = BATCH * HIST
B_PER_W = B_TOTAL // NW
ROWS_PER_GATHER = 128
GATHERS_PER_W = B_PER_W // ROWS_PER_GATHER
CHUNK_GATHERS = 10
CHUNK_ROWS = CHUNK_GATHERS * ROWS_PER_GATHER
N_CHUNKS = GATHERS_PER_W // CHUNK_GATHERS


def _make_gather():
    mesh = plsc.VectorSubcoreMesh(core_axis_name="c", subcore_axis_name="s")

    @functools.partial(
        pl.kernel,
        out_type=jax.ShapeDtypeStruct((NW, N_CHUNKS, CHUNK_ROWS, EMBED_DIM),
                                      jnp.float32),
        mesh=mesh,
        scratch_types=[
            pltpu.VMEM((GATHERS_PER_W, ROWS_PER_GATHER), jnp.int32),
            pltpu.VMEM((CHUNK_ROWS, EMBED_DIM), jnp.float32),
            pltpu.SemaphoreType.DMA,
        ],
        compiler_params=pltpu.CompilerParams(use_tc_tiling_on_sc=False),
    )
    def k(table_hbm, idx_hbm, out_hbm, idx_v, rows_v, sem):
        wid = lax.axis_index("s") * NC + lax.axis_index("c")
        pltpu.sync_copy(idx_hbm.at[wid], idx_v)

        def chunk_body(c, _):
            descs = []
            for g in range(CHUNK_GATHERS):
                d = pltpu.async_copy(
                    table_hbm.at[idx_v.at[c * CHUNK_GATHERS + g]],
                    rows_v.at[pl.ds(g * ROWS_PER_GATHER, ROWS_PER_GATHER)],
                    sem,
                )
                descs.append(d)
            for d in descs:
                d.wait()
            pltpu.sync_copy(rows_v, out_hbm.at[wid, c])
            return ()

        lax.fori_loop(0, N_CHUNKS, chunk_body, ())

    return k


_gather = _make_gather()


def kernel(inp, lengths, table):
    del lengths
    idx = (inp.astype(jnp.int32)
           .T.reshape(25, 8, 32, ROWS_PER_GATHER)
           .transpose(0, 2, 1, 3)
           .reshape(NW, GATHERS_PER_W, ROWS_PER_GATHER))
    out = _gather(table, idx)
    return (out.reshape(25, 32, 8, ROWS_PER_GATHER, EMBED_DIM)
            .transpose(0, 2, 1, 3, 4)
            .reshape(HIST, BATCH, EMBED_DIM)
            .transpose(1, 0, 2))


# speedup vs baseline: 1.0487x; 1.0487x over previous
import functools

import jax
import jax.numpy as jnp
from jax import lax
from jax.experimental import pallas as pl
from jax.experimental.pallas import tpu as pltpu
from jax.experimental.pallas import tpu_sc as plsc

VOCAB = 1000000
EMBED_DIM = 32
BATCH = 4096
HIST = 200

_INFO = plsc.get_sparse_core_info()
NC = _INFO.num_cores
NS = _INFO.num_subcores
NW = NC * NS

B_PER_W = BATCH // NW
ROWS_PER_CHUNK = 4
N_CHUNKS = B_PER_W // ROWS_PER_CHUNK


def _make_gather():
    mesh = plsc.VectorSubcoreMesh(core_axis_name="c", subcore_axis_name="s")

    @functools.partial(
        pl.kernel,
        out_type=jax.ShapeDtypeStruct((BATCH, HIST, EMBED_DIM), jnp.float32),
        mesh=mesh,
        scratch_types=[
            pltpu.VMEM((B_PER_W, HIST), jnp.int32),
            pltpu.VMEM((ROWS_PER_CHUNK, HIST, EMBED_DIM), jnp.float32),
            pltpu.SemaphoreType.DMA,
        ],
        compiler_params=pltpu.CompilerParams(use_tc_tiling_on_sc=False),
    )
    def k(table_hbm, idx_hbm, out_hbm, idx_v, rows_v, sem):
        wid = lax.axis_index("s") * NC + lax.axis_index("c")
        base = wid * B_PER_W
        pltpu.sync_copy(idx_hbm.at[pl.ds(base, B_PER_W)], idx_v)

        def chunk_body(c, _):
            descs = []
            for r in range(ROWS_PER_CHUNK):
                b = c * ROWS_PER_CHUNK + r
                descs.append(pltpu.async_copy(
                    table_hbm.at[idx_v.at[b, pl.ds(0, 128)]],
                    rows_v.at[r, pl.ds(0, 128)], sem))
                descs.append(pltpu.async_copy(
                    table_hbm.at[idx_v.at[b, pl.ds(128, HIST - 128)]],
                    rows_v.at[r, pl.ds(128, HIST - 128)], sem))
            for d in descs:
                d.wait()
            pltpu.sync_copy(rows_v,
                            out_hbm.at[pl.ds(base + c * ROWS_PER_CHUNK,
                                             ROWS_PER_CHUNK)])
            return ()

        lax.fori_loop(0, N_CHUNKS, chunk_body, ())

    return k


_gather = _make_gather()


def kernel(inp, lengths, table):
    del lengths
    return _gather(table, inp.astype(jnp.int32))


# speedup vs baseline: 1.0577x; 1.0086x over previous
import functools

import jax
import jax.numpy as jnp
from jax import lax
from jax.experimental import pallas as pl
from jax.experimental.pallas import tpu as pltpu
from jax.experimental.pallas import tpu_sc as plsc

VOCAB = 1000000
EMBED_DIM = 32
BATCH = 4096
HIST = 200

_INFO = plsc.get_sparse_core_info()
NC = _INFO.num_cores
NS = _INFO.num_subcores
NW = NC * NS

B_TOTAL = BATCH * HIST
B_PER_W = B_TOTAL // NW
ROWS_PER_GATHER = 128
GATHERS_PER_W = B_PER_W // ROWS_PER_GATHER
CHUNK_GATHERS = 10
CHUNK_ROWS = CHUNK_GATHERS * ROWS_PER_GATHER
N_CHUNKS = GATHERS_PER_W // CHUNK_GATHERS


def _make_gather():
    mesh = plsc.VectorSubcoreMesh(core_axis_name="c", subcore_axis_name="s")

    @functools.partial(
        pl.kernel,
        out_type=jax.ShapeDtypeStruct((B_TOTAL, EMBED_DIM), jnp.float32),
        mesh=mesh,
        scratch_types=[
            pltpu.VMEM((GATHERS_PER_W, ROWS_PER_GATHER), jnp.int32),
            pltpu.VMEM((CHUNK_ROWS, EMBED_DIM), jnp.float32),
            pltpu.SemaphoreType.DMA,
        ],
        compiler_params=pltpu.CompilerParams(use_tc_tiling_on_sc=False),
    )
    def k(table_hbm, idx_hbm, out_hbm, idx_v, rows_v, sem):
        wid = lax.axis_index("s") * NC + lax.axis_index("c")
        pltpu.sync_copy(idx_hbm.at[pl.ds(wid * GATHERS_PER_W, GATHERS_PER_W)],
                        idx_v)
        row_base = wid * B_PER_W

        def chunk_body(c, _):
            descs = []
            for g in range(CHUNK_GATHERS):
                descs.append(pltpu.async_copy(
                    table_hbm.at[idx_v.at[c * CHUNK_GATHERS + g]],
                    rows_v.at[pl.ds(g * ROWS_PER_GATHER, ROWS_PER_GATHER)],
                    sem))
            for d in descs:
                d.wait()
            pltpu.sync_copy(
                rows_v,
                out_hbm.at[pl.ds(row_base + c * CHUNK_ROWS, CHUNK_ROWS)])
            return ()

        lax.fori_loop(0, N_CHUNKS, chunk_body, ())

    return k


_gather = _make_gather()


def kernel(inp, lengths, table):
    del lengths
    idx = inp.astype(jnp.int32).reshape(B_TOTAL // ROWS_PER_GATHER,
                                        ROWS_PER_GATHER)
    out = _gather(table, idx)
    return out.reshape(BATCH, HIST, EMBED_DIM)


# speedup vs baseline: 1.1112x; 1.0506x over previous
import functools

import jax
import jax.numpy as jnp
from jax import lax
from jax.experimental import pallas as pl
from jax.experimental.pallas import tpu as pltpu
from jax.experimental.pallas import tpu_sc as plsc

VOCAB = 1000000
EMBED_DIM = 32
BATCH = 4096
HIST = 200

_INFO = plsc.get_sparse_core_info()
NC = _INFO.num_cores
NS = _INFO.num_subcores
NW = NC * NS

B_TOTAL = BATCH * HIST
B_PER_W = B_TOTAL // NW
ROWS_PER_GATHER = 128
GATHERS_PER_W = B_PER_W // ROWS_PER_GATHER
CHUNK_GATHERS = 10
CHUNK_ROWS = CHUNK_GATHERS * ROWS_PER_GATHER
N_CHUNKS = GATHERS_PER_W // CHUNK_GATHERS


def _make_gather():
    mesh = plsc.VectorSubcoreMesh(core_axis_name="c", subcore_axis_name="s")

    @functools.partial(
        pl.kernel,
        out_type=jax.ShapeDtypeStruct((B_TOTAL, EMBED_DIM), jnp.float32),
        mesh=mesh,
        scratch_types=[
            pltpu.VMEM((GATHERS_PER_W, ROWS_PER_GATHER), jnp.int32),
            pltpu.VMEM((CHUNK_ROWS, EMBED_DIM), jnp.float32),
            pltpu.SemaphoreType.DMA,
        ],
        compiler_params=pltpu.CompilerParams(use_tc_tiling_on_sc=False),
    )
    def k(table_hbm, idx_hbm, out_hbm, idx_v, rows_v, sem):
        wid = lax.axis_index("s") * NC + lax.axis_index("c")
        pltpu.sync_copy(idx_hbm.at[pl.ds(wid * GATHERS_PER_W, GATHERS_PER_W)],
                        idx_v)
        row_base = wid * B_PER_W

        def chunk_body(c, _):
            descs = []
            for g in range(CHUNK_GATHERS):
                descs.append(pltpu.async_copy(
                    table_hbm.at[idx_v.at[c * CHUNK_GATHERS + g]],
                    rows_v.at[pl.ds(g * ROWS_PER_GATHER, ROWS_PER_GATHER)],
                    sem))
            for d in descs:
                d.wait()
            pltpu.sync_copy(
                rows_v,
                out_hbm.at[pl.ds(row_base + c * CHUNK_ROWS, CHUNK_ROWS)])
            return ()

        lax.fori_loop(0, N_CHUNKS, chunk_body, ())

    return k


_gather = _make_gather()


def kernel(inp, lengths, table):
    del lengths
    idx = inp.astype(jnp.int32).T.reshape(B_TOTAL // ROWS_PER_GATHER,
                                          ROWS_PER_GATHER)
    out = _gather(table, idx)
    return out.reshape(HIST, BATCH, EMBED_DIM).transpose(1, 0, 2)
